# Initial kernel scaffold; baseline (speedup 1.0000x reference)
import jax
import jax.numpy as jnp
from jax.experimental import pallas as pl


def kernel(vM, eM, senders, receivers, W1m, b1m, g1m, be1m, W2m, b2m, W1n, b1n, g1n, be1n, W2n, b2n):
    raise NotImplementedError("write your pallas kernel here")


# speedup vs baseline: 1.2872x; 1.2872x over previous
import functools

import jax
import jax.numpy as jnp
from jax import lax
from jax.experimental import pallas as pl
from jax.experimental.pallas import tpu as pltpu
from jax.experimental.pallas import tpu_sc as plsc

_NC = 2
_NS = 16
_C = 128


def _mesh():
    return plsc.VectorSubcoreMesh(core_axis_name="c", subcore_axis_name="s")


def _sc_gather(table, idx2d):
    R, C = idx2d.shape
    D = table.shape[1]
    NW = _NC * _NS
    nfull, nrem = R // NW, R % NW

    @functools.partial(
        pl.kernel,
        out_type=jax.ShapeDtypeStruct((R, C, D), jnp.float32),
        mesh=_mesh(),
        scratch_types=[
            pltpu.VMEM((C,), jnp.int32),
            pltpu.VMEM((C, D), jnp.float32),
            pltpu.SemaphoreType.DMA,
        ],
    )
    def gk(table_hbm, idx_hbm, out_hbm, idx_v, rows_v, sem):
        wid = lax.axis_index("s") * _NC + lax.axis_index("c")
        nj = jnp.where(wid < nrem, nfull + 1, nfull)

        def body(j, carry):
            r = wid + j * NW
            pltpu.sync_copy(idx_hbm.at[r], idx_v)
            pltpu.async_copy(table_hbm.at[idx_v], rows_v, sem).wait()
            pltpu.sync_copy(rows_v, out_hbm.at[r])
            return carry

        lax.fori_loop(0, nj, body, 0)

    return gk(table, idx2d)


def _sc_scatter_add(e3, idx2d, Nm):
    R, C, D = e3.shape
    seg = ((Nm + 127) // 128) * 128
    rpt = seg // _NS
    nfull, nrem = R // (_NC * _NS), R % (_NC * _NS)
    zeros = jnp.zeros((seg, D), jnp.float32)

    @functools.partial(
        pl.kernel,
        out_type=jax.ShapeDtypeStruct((_NC, seg, D), jnp.float32),
        mesh=_mesh(),
        scratch_types=[
            pltpu.VMEM((C,), jnp.int32),
            pltpu.VMEM((C, D), jnp.float32),
            pltpu.SemaphoreType.DMA,
        ],
    )
    def sk(zeros_hbm, e_hbm, idx_hbm, out_hbm, idx_v, rows_v, sem):
        c = lax.axis_index("c")
        s = lax.axis_index("s")
        pltpu.sync_copy(zeros_hbm.at[pl.ds(s * rpt, rpt)],
                        out_hbm.at[c, pl.ds(s * rpt, rpt)])
        plsc.subcore_barrier()
        wid = s * _NC + c
        nj = jnp.where(wid < nrem, nfull + 1, nfull)

        def body(j, carry):
            r = wid + j * _NC * _NS
            pltpu.sync_copy(idx_hbm.at[r], idx_v)
            pltpu.sync_copy(e_hbm.at[r], rows_v)
            pltpu.sync_copy(rows_v, out_hbm.at[c].at[idx_v], add=True)
            return carry

        lax.fori_loop(0, nj, body, 0)

    out = sk(zeros, e3, idx2d)
    return out[0, :Nm], out[1, :Nm]


def _node_proj(v2, Ws, Wr):
    Nm, H = v2.shape
    NB = 1000
    grid = (Nm // NB,)

    def body(v_ref, ws_ref, wr_ref, os_ref, or_ref):
        v = v_ref[...]
        os_ref[...] = jnp.dot(v, ws_ref[...], preferred_element_type=jnp.float32)
        or_ref[...] = jnp.dot(v, wr_ref[...], preferred_element_type=jnp.float32)

    return pl.pallas_call(
        body,
        grid=grid,
        in_specs=[
            pl.BlockSpec((NB, H), lambda i: (i, 0)),
            pl.BlockSpec((H, H), lambda i: (0, 0)),
            pl.BlockSpec((H, H), lambda i: (0, 0)),
        ],
        out_specs=[
            pl.BlockSpec((NB, H), lambda i: (i, 0)),
            pl.BlockSpec((NB, H), lambda i: (i, 0)),
        ],
        out_shape=[
            jax.ShapeDtypeStruct((Nm, H), jnp.float32),
            jax.ShapeDtypeStruct((Nm, H), jnp.float32),
        ],
    )(v2, Ws, Wr)


def _mlp_ln_body(x, extra, w1_ref, w2_ref, b1_ref, g1_ref, be1_ref, b2_ref):
    pre = jnp.dot(x, w1_ref[...], preferred_element_type=jnp.float32)
    pre = pre + extra + b1_ref[...]
    h = pre * jax.nn.sigmoid(pre)
    mu = jnp.mean(h, axis=-1, keepdims=True)
    var = jnp.mean((h - mu) ** 2, axis=-1, keepdims=True)
    h = (h - mu) * lax.rsqrt(var + 1e-5) * g1_ref[...] + be1_ref[...]
    return x + jnp.dot(h, w2_ref[...], preferred_element_type=jnp.float32) + b2_ref[...]


def _edge_mlp(e2, gs, gr, W1e, W2, b1, g1, be1, b2):
    E, H = e2.shape
    EB = 640
    grid = (E // EB,)

    def body(e_ref, gs_ref, gr_ref, w1_ref, w2_ref, b1_ref, g1_ref, be1_ref,
             b2_ref, o_ref):
        o_ref[...] = _mlp_ln_body(e_ref[...], gs_ref[...] + gr_ref[...],
                                  w1_ref, w2_ref, b1_ref, g1_ref, be1_ref, b2_ref)

    row = pl.BlockSpec((EB, H), lambda i: (i, 0))
    mat = pl.BlockSpec((H, H), lambda i: (0, 0))
    vec = pl.BlockSpec((1, H), lambda i: (0, 0))
    return pl.pallas_call(
        body,
        grid=grid,
        in_specs=[row, row, row, mat, mat, vec, vec, vec, vec],
        out_specs=row,
        out_shape=jax.ShapeDtypeStruct((E, H), jnp.float32),
    )(e2, gs, gr, W1e, W2, b1, g1, be1, b2)


def _node_mlp(v2, agg0, agg1, W1v, W1a, W2, b1, g1, be1, b2):
    Nm, H = v2.shape
    NB = 1000
    grid = (Nm // NB,)

    def body(v_ref, a0_ref, a1_ref, w1_ref, w1a_ref, w2_ref, b1_ref, g1_ref,
             be1_ref, b2_ref, o_ref):
        a = a0_ref[...] + a1_ref[...]
        extra = jnp.dot(a, w1a_ref[...], preferred_element_type=jnp.float32)
        o_ref[...] = _mlp_ln_body(v_ref[...], extra, w1_ref, w2_ref, b1_ref,
                                  g1_ref, be1_ref, b2_ref)

    row = pl.BlockSpec((NB, H), lambda i: (i, 0))
    mat = pl.BlockSpec((H, H), lambda i: (0, 0))
    vec = pl.BlockSpec((1, H), lambda i: (0, 0))
    return pl.pallas_call(
        body,
        grid=grid,
        in_specs=[row, row, row, mat, mat, mat, vec, vec, vec, vec],
        out_specs=row,
        out_shape=jax.ShapeDtypeStruct((Nm, H), jnp.float32),
    )(v2, agg0, agg1, W1v, W1a, W2, b1, g1, be1, b2)


def kernel(vM, eM, senders, receivers, W1m, b1m, g1m, be1m, W2m, b2m,
           W1n, b1n, g1n, be1n, W2n, b2n):
    B, Nm, H = vM.shape
    E = eM.shape[1]
    v2 = vM[0]
    e2 = eM[0]
    sid = senders.astype(jnp.int32).reshape(E // _C, _C)
    rid = receivers.astype(jnp.int32).reshape(E // _C, _C)

    b1m_ = b1m.reshape(1, H)
    g1m_ = g1m.reshape(1, H)
    be1m_ = be1m.reshape(1, H)
    b2m_ = b2m.reshape(1, H)
    b1n_ = b1n.reshape(1, H)
    g1n_ = g1n.reshape(1, H)
    be1n_ = be1n.reshape(1, H)
    b2n_ = b2n.reshape(1, H)

    Ps, Pr = _node_proj(v2, W1m[H:2 * H], W1m[2 * H:])
    gs = _sc_gather(Ps, sid).reshape(E, H)
    gr = _sc_gather(Pr, rid).reshape(E, H)
    e2out = _edge_mlp(e2, gs, gr, W1m[:H], W2m, b1m_, g1m_, be1m_, b2m_)
    agg0, agg1 = _sc_scatter_add(e2out.reshape(E // _C, _C, H), rid, Nm)
    v2out = _node_mlp(v2, agg0, agg1, W1n[:H], W1n[H:], W2n,
                      b1n_, g1n_, be1n_, b2n_)
    return (v2out.reshape(B, Nm, H), e2out.reshape(B, E, H))


# speedup vs baseline: 1.3256x; 1.0298x over previous
import functools

import jax
import jax.numpy as jnp
from jax import lax
from jax.experimental import pallas as pl
from jax.experimental.pallas import tpu as pltpu
from jax.experimental.pallas import tpu_sc as plsc

_NC = 2
_NS = 16
_C = 128


def _mesh():
    return plsc.VectorSubcoreMesh(core_axis_name="c", subcore_axis_name="s")


def _sc_gather(table, idx2d):
    R, C = idx2d.shape
    D = table.shape[1]
    NW = _NC * _NS
    nfull, nrem = R // NW, R % NW

    @functools.partial(
        pl.kernel,
        out_type=jax.ShapeDtypeStruct((R, C, D), jnp.float32),
        mesh=_mesh(),
        scratch_types=[
            pltpu.VMEM((C,), jnp.int32),
            pltpu.VMEM((C, D), jnp.float32),
            pltpu.SemaphoreType.DMA,
        ],
    )
    def gk(table_hbm, idx_hbm, out_hbm, idx_v, rows_v, sem):
        wid = lax.axis_index("s") * _NC + lax.axis_index("c")
        nj = jnp.where(wid < nrem, nfull + 1, nfull)

        def body(j, carry):
            r = wid + j * NW
            pltpu.sync_copy(idx_hbm.at[r], idx_v)
            pltpu.async_copy(table_hbm.at[idx_v], rows_v, sem).wait()
            pltpu.sync_copy(rows_v, out_hbm.at[r])
            return carry

        lax.fori_loop(0, nj, body, 0)

    return gk(table, idx2d)


def _sc_scatter_add(e2d, rid_flat, Nm):
    E, D = e2d.shape
    NW = _NC * _NS
    own = (Nm + NW - 1) // NW
    rpt = ((own + 1 + 7) // 8) * 8
    Ep = rid_flat.shape[0]
    OUTER = Ep // 2048
    zeros = jnp.zeros((rpt, D), jnp.float32)

    @functools.partial(
        pl.kernel,
        out_type=jax.ShapeDtypeStruct((NW, rpt, D), jnp.float32),
        mesh=_mesh(),
        scratch_types=[
            pltpu.VMEM((2048,), jnp.int32),
            pltpu.VMEM((rpt, D), jnp.float32),
            pltpu.VMEM((160,), jnp.int32),
            pltpu.VMEM((16,), jnp.int32),
            pltpu.VMEM((16, D), jnp.float32),
            pltpu.SemaphoreType.DMA,
        ],
        compiler_params=pltpu.CompilerParams(needs_layout_passes=False),
    )
    def sk(zeros_hbm, e_hbm, idx_hbm, out_hbm, idxbuf, acc, plist, midx,
           grow, sem):
        c = lax.axis_index("c")
        s = lax.axis_index("s")
        wid = s * _NC + c
        base = wid * own
        iota16 = lax.iota(jnp.int32, 16)
        pltpu.sync_copy(zeros_hbm, acc)

        def drain(m):
            n16 = m // 16

            @pl.when(n16 > 0)
            def _():
                def batch(b, carry):
                    pk = plist[pl.ds(b * 16, 16)]
                    midx[...] = pk >> 9
                    pltpu.async_copy(e_hbm.at[midx], grow, sem).wait()
                    for mm in range(16):
                        li = pk[mm] & 511
                        for k in range(D // 16):
                            plsc.addupdate(acc.at[li, pl.ds(k * 16, 16)],
                                           grow[mm, pl.ds(k * 16, 16)])
                    return carry

                lax.fori_loop(0, n16, batch, 0)
                plist[pl.ds(0, 16)] = plist[pl.ds(n16 * 16, 16)]

            return m - n16 * 16

        def outer(o, m):
            pltpu.sync_copy(idx_hbm.at[pl.ds(o * 2048, 2048)], idxbuf)

            def rnd(r, m):
                for vv in range(8):
                    iv = idxbuf[pl.ds(r * 128 + vv * 16, 16)]
                    li = iv - base
                    ms = (li >= 0) & (li < own)
                    pos = (o * 2048 + r * 128 + vv * 16) + iota16
                    packed = (pos << 9) | jnp.where(ms, li, 0)
                    cs = plsc.cumsum(ms.astype(jnp.int32))
                    plsc.store_scatter(plist, [m + cs - 1], packed, mask=ms)
                    m = m + cs[15]
                return drain(m)

            return lax.fori_loop(0, 16, rnd, m)

        m = lax.fori_loop(0, OUTER, outer, jnp.int32(0))
        plist[pl.ds(m, 16)] = jnp.full((16,), own, jnp.int32)
        drain(((m + 15) // 16) * 16)
        pltpu.sync_copy(acc, out_hbm.at[wid])

    out = sk(zeros, e2d, rid_flat)
    return out[:, :own].reshape(NW * own, D)[:Nm]


def _node_proj(v2, Ws, Wr):
    Nm, H = v2.shape
    NB = 1000
    grid = (Nm // NB,)

    def body(v_ref, ws_ref, wr_ref, os_ref, or_ref):
        v = v_ref[...]
        os_ref[...] = jnp.dot(v, ws_ref[...], preferred_element_type=jnp.float32)
        or_ref[...] = jnp.dot(v, wr_ref[...], preferred_element_type=jnp.float32)

    return pl.pallas_call(
        body,
        grid=grid,
        in_specs=[
            pl.BlockSpec((NB, H), lambda i: (i, 0)),
            pl.BlockSpec((H, H), lambda i: (0, 0)),
            pl.BlockSpec((H, H), lambda i: (0, 0)),
        ],
        out_specs=[
            pl.BlockSpec((NB, H), lambda i: (i, 0)),
            pl.BlockSpec((NB, H), lambda i: (i, 0)),
        ],
        out_shape=[
            jax.ShapeDtypeStruct((Nm, H), jnp.float32),
            jax.ShapeDtypeStruct((Nm, H), jnp.float32),
        ],
    )(v2, Ws, Wr)


def _mlp_ln_body(x, extra, w1_ref, w2_ref, b1_ref, g1_ref, be1_ref, b2_ref):
    pre = jnp.dot(x, w1_ref[...], preferred_element_type=jnp.float32)
    pre = pre + extra + b1_ref[...]
    h = pre * jax.nn.sigmoid(pre)
    mu = jnp.mean(h, axis=-1, keepdims=True)
    var = jnp.mean((h - mu) ** 2, axis=-1, keepdims=True)
    h = (h - mu) * lax.rsqrt(var + 1e-5) * g1_ref[...] + be1_ref[...]
    return x + jnp.dot(h, w2_ref[...], preferred_element_type=jnp.float32) + b2_ref[...]


def _edge_mlp(e2, gs, gr, W1e, W2, b1, g1, be1, b2):
    E, H = e2.shape
    EB = 640
    grid = (E // EB,)

    def body(e_ref, gs_ref, gr_ref, w1_ref, w2_ref, b1_ref, g1_ref, be1_ref,
             b2_ref, o_ref):
        o_ref[...] = _mlp_ln_body(e_ref[...], gs_ref[...] + gr_ref[...],
                                  w1_ref, w2_ref, b1_ref, g1_ref, be1_ref, b2_ref)

    row = pl.BlockSpec((EB, H), lambda i: (i, 0))
    mat = pl.BlockSpec((H, H), lambda i: (0, 0))
    vec = pl.BlockSpec((1, H), lambda i: (0, 0))
    return pl.pallas_call(
        body,
        grid=grid,
        in_specs=[row, row, row, mat, mat, vec, vec, vec, vec],
        out_specs=row,
        out_shape=jax.ShapeDtypeStruct((E, H), jnp.float32),
    )(e2, gs, gr, W1e, W2, b1, g1, be1, b2)


def _node_mlp(v2, agg, W1v, W1a, W2, b1, g1, be1, b2):
    Nm, H = v2.shape
    NB = 1000
    grid = (Nm // NB,)

    def body(v_ref, a_ref, w1_ref, w1a_ref, w2_ref, b1_ref, g1_ref,
             be1_ref, b2_ref, o_ref):
        extra = jnp.dot(a_ref[...], w1a_ref[...],
                        preferred_element_type=jnp.float32)
        o_ref[...] = _mlp_ln_body(v_ref[...], extra, w1_ref, w2_ref, b1_ref,
                                  g1_ref, be1_ref, b2_ref)

    row = pl.BlockSpec((NB, H), lambda i: (i, 0))
    mat = pl.BlockSpec((H, H), lambda i: (0, 0))
    vec = pl.BlockSpec((1, H), lambda i: (0, 0))
    return pl.pallas_call(
        body,
        grid=grid,
        in_specs=[row, row, mat, mat, mat, vec, vec, vec, vec],
        out_specs=row,
        out_shape=jax.ShapeDtypeStruct((Nm, H), jnp.float32),
    )(v2, agg, W1v, W1a, W2, b1, g1, be1, b2)


def kernel(vM, eM, senders, receivers, W1m, b1m, g1m, be1m, W2m, b2m,
           W1n, b1n, g1n, be1n, W2n, b2n):
    B, Nm, H = vM.shape
    E = eM.shape[1]
    v2 = vM[0]
    e2 = eM[0]
    sid = senders.astype(jnp.int32).reshape(E // _C, _C)
    rid = receivers.astype(jnp.int32).reshape(E // _C, _C)

    b1m_ = b1m.reshape(1, H)
    g1m_ = g1m.reshape(1, H)
    be1m_ = be1m.reshape(1, H)
    b2m_ = b2m.reshape(1, H)
    b1n_ = b1n.reshape(1, H)
    g1n_ = g1n.reshape(1, H)
    be1n_ = be1n.reshape(1, H)
    b2n_ = b2n.reshape(1, H)

    Ps, Pr = _node_proj(v2, W1m[H:2 * H], W1m[2 * H:])
    gs = _sc_gather(Ps, sid).reshape(E, H)
    gr = _sc_gather(Pr, rid).reshape(E, H)
    e2out = _edge_mlp(e2, gs, gr, W1m[:H], W2m, b1m_, g1m_, be1m_, b2m_)
    r32 = receivers.astype(jnp.int32)
    Ep = ((E + 2047) // 2048) * 2048
    rid_flat = jnp.concatenate(
        [r32, jnp.full((Ep - E,), 2 ** 20, jnp.int32)])
    agg = _sc_scatter_add(e2out, rid_flat, Nm)
    v2out = _node_mlp(v2, agg, W1n[:H], W1n[H:], W2n,
                      b1n_, g1n_, be1n_, b2n_)
    return (v2out.reshape(B, Nm, H), e2out.reshape(B, E, H))
